# Initial kernel scaffold; baseline (speedup 1.0000x reference)
import jax
import jax.numpy as jnp
from jax.experimental import pallas as pl


def kernel(x, LUT):
    raise NotImplementedError("write your pallas kernel here")


# speedup vs baseline: 225.2203x; 225.2203x over previous
import functools

import jax
import jax.numpy as jnp
from jax import lax
from jax.experimental import pallas as pl
from jax.experimental.pallas import tpu as pltpu
from jax.experimental.pallas import tpu_sc as plsc

DIM = 33
NLUT = DIM * DIM * DIM
B, H, W = 8, 512, 512
NPIX = H * W
NW = 32
TILES_PER_IMG = NW // B
PIX_PER_TILE = NPIX // TILES_PER_IMG
CHUNK = 2048
NCHUNK = PIX_PER_TILE // CHUNK
GROUPS = CHUNK // 16

_CORNER_OFF = (0, 1, DIM, DIM + 1, DIM * DIM, DIM * DIM + 1,
               DIM * DIM + DIM, DIM * DIM + DIM + 1)


def _body(x_hbm, l0_hbm, l1_hbm, l2_hbm, out_hbm,
          lut0, lut1, lut2, xr, xg, xb, yr, yg, yb):
    pltpu.sync_copy(l0_hbm, lut0)
    pltpu.sync_copy(l1_hbm, lut1)
    pltpu.sync_copy(l2_hbm, lut2)

    wid = lax.axis_index("s") * 2 + lax.axis_index("c")
    img = wid // TILES_PER_IMG
    quarter = wid % TILES_PER_IMG
    p0 = img * (3 * NPIX) + quarter * PIX_PER_TILE

    def chunk_body(ci, _):
        off = p0 + ci * CHUNK
        pltpu.sync_copy(x_hbm.at[pl.ds(off, CHUNK)], xr)
        pltpu.sync_copy(x_hbm.at[pl.ds(off + NPIX, CHUNK)], xg)
        pltpu.sync_copy(x_hbm.at[pl.ds(off + 2 * NPIX, CHUNK)], xb)

        def group_body(gi, _):
            s = gi * 16
            r = xr[pl.ds(s, 16)]
            g = xg[pl.ds(s, 16)]
            b = xb[pl.ds(s, 16)]
            rf = r * jnp.float32(DIM - 1)
            gf = g * jnp.float32(DIM - 1)
            bf = b * jnp.float32(DIM - 1)
            ri = jnp.minimum(rf.astype(jnp.int32), DIM - 2)
            gi_ = jnp.minimum(gf.astype(jnp.int32), DIM - 2)
            bi = jnp.minimum(bf.astype(jnp.int32), DIM - 2)
            rd = rf - ri.astype(jnp.float32)
            gd = gf - gi_.astype(jnp.float32)
            bd = bf - bi.astype(jnp.float32)
            base = ri + gi_ * DIM + bi * (DIM * DIM)
            omr = 1.0 - rd
            omg = 1.0 - gd
            omb = 1.0 - bd
            a00 = omr * omg
            a10 = rd * omg
            a01 = omr * gd
            a11 = rd * gd
            ws = (a00 * omb, a10 * omb, a01 * omb, a11 * omb,
                  a00 * bd, a10 * bd, a01 * bd, a11 * bd)
            acc0 = jnp.zeros((16,), jnp.float32)
            acc1 = jnp.zeros((16,), jnp.float32)
            acc2 = jnp.zeros((16,), jnp.float32)
            for k in range(8):
                idx = base + _CORNER_OFF[k]
                acc0 = acc0 + ws[k] * plsc.load_gather(lut0, [idx])
                acc1 = acc1 + ws[k] * plsc.load_gather(lut1, [idx])
                acc2 = acc2 + ws[k] * plsc.load_gather(lut2, [idx])
            yr[pl.ds(s, 16)] = acc0
            yg[pl.ds(s, 16)] = acc1
            yb[pl.ds(s, 16)] = acc2
            return 0

        lax.fori_loop(0, GROUPS, group_body, 0)
        pltpu.sync_copy(yr, out_hbm.at[pl.ds(off, CHUNK)])
        pltpu.sync_copy(yg, out_hbm.at[pl.ds(off + NPIX, CHUNK)])
        pltpu.sync_copy(yb, out_hbm.at[pl.ds(off + 2 * NPIX, CHUNK)])
        return 0

    lax.fori_loop(0, NCHUNK, chunk_body, 0)


def kernel(x, LUT):
    lut_flat = LUT.reshape(3, NLUT)
    k = functools.partial(
        pl.kernel,
        out_type=jax.ShapeDtypeStruct((B * 3 * NPIX,), jnp.float32),
        mesh=plsc.VectorSubcoreMesh(core_axis_name="c", subcore_axis_name="s"),
        compiler_params=pltpu.CompilerParams(needs_layout_passes=False),
        scratch_types=[
            pltpu.VMEM((NLUT,), jnp.float32),
            pltpu.VMEM((NLUT,), jnp.float32),
            pltpu.VMEM((NLUT,), jnp.float32),
            pltpu.VMEM((CHUNK,), jnp.float32),
            pltpu.VMEM((CHUNK,), jnp.float32),
            pltpu.VMEM((CHUNK,), jnp.float32),
            pltpu.VMEM((CHUNK,), jnp.float32),
            pltpu.VMEM((CHUNK,), jnp.float32),
            pltpu.VMEM((CHUNK,), jnp.float32),
        ],
    )(_body)
    out = k(x.reshape(-1), lut_flat[0], lut_flat[1], lut_flat[2])
    return out.reshape(B, 3, H, W)
